# baseline (speedup 1.0000x reference)
import functools

import jax
import jax.numpy as jnp
import numpy as np
from jax.experimental import pallas as pl
from jax.experimental.pallas import tpu as pltpu
from jax.experimental.pallas import tpu_sc as plsc

_B = 1024
_D = 64
_N = 100000
_K = 3
_L = 2
_NL = 12

_TN = 2000
_NT = (_N + _TN - 1) // _TN
_NEG = -3.0e38
_FBIG = 3.0e38

_NC = 2
_NS = 16
_NW = _NC * _NS
_BK = _B * _K
_GW = _BK // _NW
_DP = 128


def _topk_body(q_ref, db_ref, lab_ref, s_ref, i_ref, p_ref):
    pid = pl.program_id(0)

    @pl.when(pid == 0)
    def _init():
        s_ref[...] = jnp.full(s_ref.shape, _NEG, jnp.float32)
        i_ref[...] = jnp.zeros(i_ref.shape, jnp.float32)

    qn = q_ref[...]
    fn = db_ref[...]
    p_ref[:, 0:_D] = fn.astype(jnp.float32)
    p_ref[:, _D:_D + 1] = lab_ref[...]
    sims = jax.lax.dot_general(qn, fn, (((1,), (1,)), ((), ())),
                               preferred_element_type=jnp.float32)
    base = jax.lax.convert_element_type(pid * _TN, jnp.float32)
    colid = jax.lax.broadcasted_iota(
        jnp.int32, sims.shape, 1).astype(jnp.float32) + base

    for j in range(_K):
        m = jnp.max(sims, axis=1, keepdims=True)
        gi = jnp.min(jnp.where(sims == m, colid, _FBIG), axis=1, keepdims=True)
        if j < _K - 1:
            sims = jnp.where(colid == gi, _NEG, sims)
        s0 = s_ref[:, 0:1]
        s1 = s_ref[:, 1:2]
        s2 = s_ref[:, 2:3]
        i0 = i_ref[:, 0:1]
        i1 = i_ref[:, 1:2]
        i2 = i_ref[:, 2:3]
        b0 = m > s0
        b1 = m > s1
        b2 = m > s2
        s_ref[:, 0:1] = jnp.where(b0, m, s0)
        s_ref[:, 1:2] = jnp.where(b0, s0, jnp.where(b1, m, s1))
        s_ref[:, 2:3] = jnp.where(b1, s1, jnp.where(b2, m, s2))
        i_ref[:, 0:1] = jnp.where(b0, gi, i0)
        i_ref[:, 1:2] = jnp.where(b0, i0, jnp.where(b1, gi, i1))
        i_ref[:, 2:3] = jnp.where(b1, i1, jnp.where(b2, gi, i2))


def _topk_call(qn, featsn, labels2d):
    return pl.pallas_call(
        _topk_body,
        grid=(_NT,),
        in_specs=[
            pl.BlockSpec((_B, _D), lambda i: (0, 0)),
            pl.BlockSpec((_TN, _D), lambda i: (i, 0)),
            pl.BlockSpec((_TN, 1), lambda i: (i, 0)),
        ],
        out_specs=[
            pl.BlockSpec((_B, 8), lambda i: (0, 0)),
            pl.BlockSpec((_B, 8), lambda i: (0, 0)),
            pl.BlockSpec((_TN, _DP), lambda i: (i, 0)),
        ],
        out_shape=[
            jax.ShapeDtypeStruct((_B, 8), jnp.float32),
            jax.ShapeDtypeStruct((_B, 8), jnp.float32),
            jax.ShapeDtypeStruct((_NT * _TN, _DP), jnp.float32),
        ],
    )(qn, featsn, labels2d)


def _gather_body(tbl_hbm, idx_hbm, out_hbm, idx_v, rows_v, sem):
    wid = jax.lax.axis_index("s") * _NC + jax.lax.axis_index("c")
    base = wid * _GW
    pltpu.sync_copy(idx_hbm.at[pl.ds(base, _GW)], idx_v)
    pltpu.async_copy(tbl_hbm.at[idx_v], rows_v, sem).wait()
    pltpu.sync_copy(rows_v, out_hbm.at[pl.ds(base, _GW)])


def _gather_call(table, idx):
    mesh = plsc.VectorSubcoreMesh(core_axis_name="c", subcore_axis_name="s")
    k = functools.partial(
        pl.kernel,
        mesh=mesh,
        out_type=jax.ShapeDtypeStruct((_BK, _DP), jnp.float32),
        scratch_types=[
            pltpu.VMEM((_GW,), jnp.int32),
            pltpu.VMEM((_GW, _DP), jnp.float32),
            pltpu.SemaphoreType.DMA,
        ],
    )(_gather_body)
    return k(table, idx)


def _mea_body(hx_ref, g_ref, sc_ref, wq_ref, wk_ref, wv_ref, wo_ref,
              bq_ref, bk_ref, bv_ref, bo_ref, dw_ref, db_ref, ow_ref,
              ob_ref, out_ref):
    g = g_ref[...]
    fn = g[:, 0:_D]
    lb = g[:, _D:_D + 1]
    lbi = lb.astype(jnp.int32)
    sc = sc_ref[...]
    iota64 = jax.lax.broadcasted_iota(jnp.int32, (_BK, _D), 1)
    cls = jnp.where(iota64 == lbi, 1.0, 0.0).astype(jnp.float32) * sc
    h = [cls, hx_ref[...], fn]

    def mm(a, w):
        return jax.lax.dot_general(a.astype(jnp.bfloat16),
                                   w.astype(jnp.bfloat16),
                                   (((1,), (0,)), ((), ())),
                                   preferred_element_type=jnp.float32)

    def tb(x):
        return x.astype(jnp.bfloat16).astype(jnp.float32)

    for i in range(_L):
        wq = wq_ref[i]
        wk = wk_ref[i]
        wv = wv_ref[i]
        wo = wo_ref[i]
        bq = bq_ref[i]
        bk = bk_ref[i]
        bv = bv_ref[i]
        bo = bo_ref[i]
        qs = [mm(t, wq) + bq for t in h]
        ks = [mm(t, wk) + bk for t in h]
        vs = [mm(t, wv) + bv for t in h]
        qs = [tb(t) for t in qs]
        ks = [tb(t) for t in ks]
        vbs = [tb(t) for t in vs]
        newh = []
        for s in range(3):
            e = [jnp.sum(qs[s] * ks[t], axis=1, keepdims=True) * 0.25
                 for t in range(3)]
            m = jnp.maximum(jnp.maximum(e[0], e[1]), e[2])
            x = [jnp.exp(et - m) for et in e]
            den = x[0] + x[1] + x[2]
            a = [tb(xt / den) for xt in x]
            o = a[0] * vbs[0] + a[1] * vbs[1] + a[2] * vbs[2]
            newh.append(mm(o, wo) + bo)
        h = newh

    x = jnp.tanh(mm(h[0], dw_ref[...]) + db_ref[...])
    lg = mm(x, ow_ref[...]) + ob_ref[...]
    iota128 = jax.lax.broadcasted_iota(jnp.int32, (_BK, 128), 1)
    oh = jnp.where(iota128 == lbi, 1.0, 0.0).astype(jnp.float32)
    comb = 0.5 * lg + 0.5 * oh
    r = comb.reshape(_B, _K, 128)
    out_ref[...] = jnp.sum(r, axis=1) * (1.0 / _K)


def _mea_call(hx, g, sc3, wq, wk, wv, wo, bq, bk, bv, bo, dw, dbias, ow, ob):
    return pl.pallas_call(
        _mea_body,
        out_shape=jax.ShapeDtypeStruct((_B, 128), jnp.float32),
    )(hx, g, sc3, wq, wk, wv, wo, bq, bk, bv, bo, dw, dbias, ow, ob)


def kernel(queries, database, WQ, bQ, WK, bK, WV, bV, WO, bO,
           dense_W, dense_b, out_W, out_b):
    feats = database[:, :_D]
    fnb = (feats / jnp.linalg.norm(feats, axis=-1, keepdims=True)
           ).astype(jnp.bfloat16)
    labels2d = database[:, _D:]
    qnb = (queries / jnp.linalg.norm(queries, axis=-1, keepdims=True)
           ).astype(jnp.bfloat16)
    s_out, i_out, dbp = _topk_call(qnb, fnb, labels2d)
    idx = i_out[:, :_K].astype(jnp.int32).reshape(-1)
    g = _gather_call(dbp, idx)
    hx = jnp.broadcast_to(queries[:, None, :], (_B, _K, _D)).reshape(_BK, _D)
    sc3 = s_out[:, :_K].reshape(_BK, 1)
    wq = jnp.swapaxes(WQ, 1, 2)
    wk = jnp.swapaxes(WK, 1, 2)
    wv = jnp.swapaxes(WV, 1, 2)
    wo = jnp.swapaxes(WO, 1, 2)
    bq = bQ.reshape(_L, 1, _D)
    bk = bK.reshape(_L, 1, _D)
    bv = bV.reshape(_L, 1, _D)
    bo = bO.reshape(_L, 1, _D)
    dw = dense_W.T
    dbias = dense_b.reshape(1, _D)
    ow = jnp.zeros((_D, 128), jnp.float32).at[:, :_NL].set(out_W.T)
    ob = jnp.zeros((1, 128), jnp.float32).at[0, :_NL].set(out_b)
    full = _mea_call(hx, g, sc3, wq, wk, wv, wo, bq, bk, bv, bo,
                     dw, dbias, ow, ob)
    return full[:, :_NL]


# speedup vs baseline: 1.0234x; 1.0234x over previous
import functools

import jax
import jax.numpy as jnp
import numpy as np
from jax.experimental import pallas as pl
from jax.experimental.pallas import tpu as pltpu
from jax.experimental.pallas import tpu_sc as plsc

_B = 1024
_D = 64
_N = 100000
_K = 3
_L = 2
_NL = 12

_TN = 2000
_NT = (_N + _TN - 1) // _TN
_NEG = -3.0e38
_FBIG = 3.0e38

_NC = 2
_NS = 16
_NW = _NC * _NS
_BK = _B * _K
_GW = _BK // _NW
_DP = 128


def _topk_body(q_ref, db_ref, nrm_ref, s_ref, i_ref, p_ref):
    pid = pl.program_id(0)

    @pl.when(pid == 0)
    def _init():
        s_ref[...] = jnp.full(s_ref.shape, _NEG, jnp.float32)
        i_ref[...] = jnp.zeros(i_ref.shape, jnp.float32)

    qn = q_ref[...]
    fn = db_ref[:, 0:_D] / nrm_ref[...]
    p_ref[:, 0:_D] = fn
    p_ref[:, _D:_D + 1] = db_ref[:, _D:_D + 1]
    sims = jax.lax.dot_general(qn.astype(jnp.bfloat16),
                               fn.astype(jnp.bfloat16),
                               (((1,), (1,)), ((), ())),
                               preferred_element_type=jnp.float32)
    base = jax.lax.convert_element_type(pid * _TN, jnp.float32)
    colid = jax.lax.broadcasted_iota(
        jnp.int32, sims.shape, 1).astype(jnp.float32) + base

    for j in range(_K):
        m = jnp.max(sims, axis=1, keepdims=True)
        gi = jnp.min(jnp.where(sims == m, colid, _FBIG), axis=1, keepdims=True)
        if j < _K - 1:
            sims = jnp.where(colid == gi, _NEG, sims)
        s0 = s_ref[:, 0:1]
        s1 = s_ref[:, 1:2]
        s2 = s_ref[:, 2:3]
        i0 = i_ref[:, 0:1]
        i1 = i_ref[:, 1:2]
        i2 = i_ref[:, 2:3]
        b0 = m > s0
        b1 = m > s1
        b2 = m > s2
        s_ref[:, 0:1] = jnp.where(b0, m, s0)
        s_ref[:, 1:2] = jnp.where(b0, s0, jnp.where(b1, m, s1))
        s_ref[:, 2:3] = jnp.where(b1, s1, jnp.where(b2, m, s2))
        i_ref[:, 0:1] = jnp.where(b0, gi, i0)
        i_ref[:, 1:2] = jnp.where(b0, i0, jnp.where(b1, gi, i1))
        i_ref[:, 2:3] = jnp.where(b1, i1, jnp.where(b2, gi, i2))


def _topk_call(qn, database, norms):
    return pl.pallas_call(
        _topk_body,
        grid=(_NT,),
        in_specs=[
            pl.BlockSpec((_B, _D), lambda i: (0, 0)),
            pl.BlockSpec((_TN, _D + 1), lambda i: (i, 0)),
            pl.BlockSpec((_TN, 1), lambda i: (i, 0)),
        ],
        out_specs=[
            pl.BlockSpec((_B, 8), lambda i: (0, 0)),
            pl.BlockSpec((_B, 8), lambda i: (0, 0)),
            pl.BlockSpec((_TN, _DP), lambda i: (i, 0)),
        ],
        out_shape=[
            jax.ShapeDtypeStruct((_B, 8), jnp.float32),
            jax.ShapeDtypeStruct((_B, 8), jnp.float32),
            jax.ShapeDtypeStruct((_NT * _TN, _DP), jnp.float32),
        ],
    )(qn, database, norms)


def _gather_body(tbl_hbm, idx_hbm, out_hbm, idx_v, rows_v, sem):
    wid = jax.lax.axis_index("s") * _NC + jax.lax.axis_index("c")
    base = wid * _GW
    pltpu.sync_copy(idx_hbm.at[pl.ds(base, _GW)], idx_v)
    pltpu.async_copy(tbl_hbm.at[idx_v], rows_v, sem).wait()
    pltpu.sync_copy(rows_v, out_hbm.at[pl.ds(base, _GW)])


def _gather_call(table, idx):
    mesh = plsc.VectorSubcoreMesh(core_axis_name="c", subcore_axis_name="s")
    k = functools.partial(
        pl.kernel,
        mesh=mesh,
        out_type=jax.ShapeDtypeStruct((_BK, _DP), jnp.float32),
        scratch_types=[
            pltpu.VMEM((_GW,), jnp.int32),
            pltpu.VMEM((_GW, _DP), jnp.float32),
            pltpu.SemaphoreType.DMA,
        ],
    )(_gather_body)
    return k(table, idx)


def _mea_body(hx_ref, g_ref, sc_ref, wq_ref, wk_ref, wv_ref, wo_ref,
              bq_ref, bk_ref, bv_ref, bo_ref, dw_ref, db_ref, ow_ref,
              ob_ref, out_ref):
    g = g_ref[...]
    fn = g[:, 0:_D]
    lb = g[:, _D:_D + 1]
    lbi = lb.astype(jnp.int32)
    sc = sc_ref[...]
    iota64 = jax.lax.broadcasted_iota(jnp.int32, (_BK, _D), 1)
    cls = jnp.where(iota64 == lbi, 1.0, 0.0).astype(jnp.float32) * sc
    h = [cls, hx_ref[...], fn]

    def mm(a, w):
        return jax.lax.dot_general(a.astype(jnp.bfloat16),
                                   w.astype(jnp.bfloat16),
                                   (((1,), (0,)), ((), ())),
                                   preferred_element_type=jnp.float32)

    def tb(x):
        return x.astype(jnp.bfloat16).astype(jnp.float32)

    for i in range(_L):
        wq = wq_ref[i]
        wk = wk_ref[i]
        wv = wv_ref[i]
        wo = wo_ref[i]
        bq = bq_ref[i]
        bk = bk_ref[i]
        bv = bv_ref[i]
        bo = bo_ref[i]
        qs = [mm(t, wq) + bq for t in h]
        ks = [mm(t, wk) + bk for t in h]
        vs = [mm(t, wv) + bv for t in h]
        qs = [tb(t) for t in qs]
        ks = [tb(t) for t in ks]
        vbs = [tb(t) for t in vs]
        newh = []
        for s in range(3):
            e = [jnp.sum(qs[s] * ks[t], axis=1, keepdims=True) * 0.25
                 for t in range(3)]
            m = jnp.maximum(jnp.maximum(e[0], e[1]), e[2])
            x = [jnp.exp(et - m) for et in e]
            den = x[0] + x[1] + x[2]
            a = [tb(xt / den) for xt in x]
            o = a[0] * vbs[0] + a[1] * vbs[1] + a[2] * vbs[2]
            newh.append(mm(o, wo) + bo)
        h = newh

    x = jnp.tanh(mm(h[0], dw_ref[...]) + db_ref[...])
    lg = mm(x, ow_ref[...]) + ob_ref[...]
    iota128 = jax.lax.broadcasted_iota(jnp.int32, (_BK, 128), 1)
    oh = jnp.where(iota128 == lbi, 1.0, 0.0).astype(jnp.float32)
    comb = 0.5 * lg + 0.5 * oh
    r = comb.reshape(_B, _K, 128)
    out_ref[...] = jnp.sum(r, axis=1) * (1.0 / _K)


def _mea_call(hx, g, sc3, wq, wk, wv, wo, bq, bk, bv, bo, dw, dbias, ow, ob):
    return pl.pallas_call(
        _mea_body,
        out_shape=jax.ShapeDtypeStruct((_B, 128), jnp.float32),
    )(hx, g, sc3, wq, wk, wv, wo, bq, bk, bv, bo, dw, dbias, ow, ob)


def kernel(queries, database, WQ, bQ, WK, bK, WV, bV, WO, bO,
           dense_W, dense_b, out_W, out_b):
    norms = jnp.linalg.norm(database[:, :_D], axis=-1, keepdims=True)
    qn = queries / jnp.linalg.norm(queries, axis=-1, keepdims=True)
    s_out, i_out, dbp = _topk_call(qn, database, norms)
    idx = i_out[:, :_K].astype(jnp.int32).reshape(-1)
    g = _gather_call(dbp, idx)
    hx = jnp.broadcast_to(queries[:, None, :], (_B, _K, _D)).reshape(_BK, _D)
    sc3 = s_out[:, :_K].reshape(_BK, 1)
    wq = jnp.swapaxes(WQ, 1, 2)
    wk = jnp.swapaxes(WK, 1, 2)
    wv = jnp.swapaxes(WV, 1, 2)
    wo = jnp.swapaxes(WO, 1, 2)
    bq = bQ.reshape(_L, 1, _D)
    bk = bK.reshape(_L, 1, _D)
    bv = bV.reshape(_L, 1, _D)
    bo = bO.reshape(_L, 1, _D)
    dw = dense_W.T
    dbias = dense_b.reshape(1, _D)
    ow = jnp.zeros((_D, 128), jnp.float32).at[:, :_NL].set(out_W.T)
    ob = jnp.zeros((1, 128), jnp.float32).at[0, :_NL].set(out_b)
    full = _mea_call(hx, g, sc3, wq, wk, wv, wo, bq, bk, bv, bo,
                     dw, dbias, ow, ob)
    return full[:, :_NL]
